# baseline (speedup 1.0000x reference)
import functools

import jax
import jax.numpy as jnp
from jax import lax
from jax.experimental import pallas as pl
from jax.experimental.pallas import tpu as pltpu
from jax.experimental.pallas import tpu_sc as plsc

_NUM_ENTITIES = 1000000
_NUM_RELATIONS = 1000
_EMB_DIM = 64
_BATCH = 16384
_L = 16
_NC, _NS = 2, 16
_NW = _NC * _NS
_BPW = _BATCH // _NW
_CH = 16
_NCH = _BPW // _CH
_NP = _NCH // 2


def _rsqrt_nr(s):
    i = plsc.bitcast(s, jnp.int32)
    i = jnp.int32(0x5F3759DF) - lax.shift_right_logical(i, 1)
    y = plsc.bitcast(i, jnp.float32)
    for _ in range(2):
        y = y * (jnp.float32(1.5) - jnp.float32(0.5) * s * y * y)
    return y


def _make_sc_call():
    mesh = plsc.VectorSubcoreMesh(core_axis_name="c", subcore_axis_name="s")
    out = jax.ShapeDtypeStruct((_EMB_DIM, _BATCH), jnp.float32)

    @functools.partial(
        pl.kernel,
        mesh=mesh,
        out_type=[out, out, out],
        compiler_params=pltpu.CompilerParams(needs_layout_passes=False),
        scratch_types=[
            pltpu.VMEM((4, 128), jnp.int32),
            pltpu.VMEM((4, 128), jnp.int32),
            pltpu.VMEM((4, 128), jnp.int32),
            pltpu.VMEM((4, 128), jnp.int32),
            pltpu.VMEM((4, 128), jnp.int32),
            pltpu.VMEM((4, 128), jnp.int32),
            pltpu.VMEM((_CH, 8, 128), jnp.float32),
            pltpu.VMEM((_CH, 8, 128), jnp.float32),
            pltpu.VMEM((_EMB_DIM, _BPW), jnp.float32),
            pltpu.SemaphoreType.DMA,
            pltpu.SemaphoreType.DMA,
            pltpu.SemaphoreType.DMA,
        ],
    )
    def call(ent_hbm, rel_hbm, ph_hbm, pr_hbm, pt_hbm,
             h_out, r_out, t_out,
             raw_h, raw_r, raw_t, b8_h, b8_r, b8_t,
             blk0, blk1, outbuf, sem0, sem1, sem_out):
        wid = lax.axis_index("s") * _NC + lax.axis_index("c")
        base = wid * _BPW
        iota = jnp.arange(_L, dtype=jnp.int32)
        cols = [jnp.full((_L,), c, jnp.int32) for c in range(_EMB_DIM)]

        for pos_hbm, raw, b8 in ((ph_hbm, raw_h, b8_h),
                                 (pt_hbm, raw_t, b8_t),
                                 (pr_hbm, raw_r, b8_r)):
            for j in range(4):
                pltpu.sync_copy(pos_hbm.at[pl.ds(base + j * 128, 128)],
                                raw.at[j])
            for j in range(4):
                for o in range(0, 128, _L):
                    b8[j, pl.ds(o, _L)] = lax.shift_right_logical(
                        raw[j, pl.ds(o, _L)], 3)

        def fire(tbl, b8, c, blk, sem):
            j = lax.shift_right_logical(c, 3)
            o = (c & 7) * _L
            b8_vec = b8[j, pl.ds(o, _L)]
            for k in range(_CH):
                pltpu.async_copy(tbl.at[b8_vec[k]], blk.at[k], sem)

        def drain(tbl, blk, sem):
            pltpu.make_async_copy(tbl.at[pl.ds(0, _CH)], blk, sem).wait()

        def process(raw, c, blk, normalize):
            j = lax.shift_right_logical(c, 3)
            o = (c & 7) * _L
            idx_vec = raw[j, pl.ds(o, _L)]
            wr = idx_vec & 7
            dst = pl.ds(c * _L, _L)
            if normalize:
                acc0 = jnp.zeros((_L,), jnp.float32)
                acc1 = jnp.zeros((_L,), jnp.float32)
                acc2 = jnp.zeros((_L,), jnp.float32)
                acc3 = jnp.zeros((_L,), jnp.float32)
                for c4 in range(0, _EMB_DIM, 4):
                    v0 = plsc.load_gather(blk, [iota, wr, cols[c4]])
                    v1 = plsc.load_gather(blk, [iota, wr, cols[c4 + 1]])
                    v2 = plsc.load_gather(blk, [iota, wr, cols[c4 + 2]])
                    v3 = plsc.load_gather(blk, [iota, wr, cols[c4 + 3]])
                    acc0 = acc0 + v0 * v0
                    acc1 = acc1 + v1 * v1
                    acc2 = acc2 + v2 * v2
                    acc3 = acc3 + v3 * v3
                s = (acc0 + acc1) + (acc2 + acc3)
                y = _rsqrt_nr(s)
                y = jnp.where(idx_vec == _NUM_ENTITIES - 1,
                              jnp.float32(1.0), y)
                for cc in range(_EMB_DIM):
                    v = plsc.load_gather(blk, [iota, wr, cols[cc]])
                    outbuf[cc, dst] = v * y
            else:
                for cc in range(_EMB_DIM):
                    v = plsc.load_gather(blk, [iota, wr, cols[cc]])
                    outbuf[cc, dst] = v

        def run_tensor(tbl, raw, b8, normalize, pre=None):
            fire(tbl, b8, 0, blk0, sem0)
            fire(tbl, b8, 1, blk1, sem1)
            if pre is not None:
                pre.wait()

            def body(p, carry):
                c = p * 2
                drain(tbl, blk0, sem0)
                process(raw, c, blk0, normalize)

                @pl.when(p < _NP - 1)
                def _():
                    fire(tbl, b8, c + 2, blk0, sem0)

                drain(tbl, blk1, sem1)
                process(raw, c + 1, blk1, normalize)

                @pl.when(p < _NP - 1)
                def _():
                    fire(tbl, b8, c + 3, blk1, sem1)

                return carry

            lax.fori_loop(0, _NP, body, 0, unroll=False)

        out_slice = pl.ds(base, _BPW)
        run_tensor(ent_hbm, raw_h, b8_h, True)
        cp_h = pltpu.async_copy(outbuf, h_out.at[:, out_slice], sem_out)
        run_tensor(ent_hbm, raw_t, b8_t, True, pre=cp_h)
        cp_t = pltpu.async_copy(outbuf, t_out.at[:, out_slice], sem_out)
        run_tensor(rel_hbm, raw_r, b8_r, False, pre=cp_t)
        cp_r = pltpu.async_copy(outbuf, r_out.at[:, out_slice], sem_out)
        cp_r.wait()

    return call


_sc_call = _make_sc_call()


def kernel(entity_embds, rel_embds, pos_h, pos_r, pos_t):
    entp = jnp.pad(entity_embds, ((0, 0), (0, 64)))
    relp = jnp.pad(rel_embds, ((0, 0), (0, 64)))
    ent3 = jnp.reshape(entp, (_NUM_ENTITIES // 8, 8, 128))
    rel3 = jnp.reshape(relp, (_NUM_RELATIONS // 8, 8, 128))
    hT, rT, tT = _sc_call(ent3, rel3,
                          pos_h.astype(jnp.int32),
                          pos_r.astype(jnp.int32),
                          pos_t.astype(jnp.int32))
    return (hT.T, rT.T, tT.T)


# speedup vs baseline: 1.8120x; 1.8120x over previous
import functools

import jax
import jax.numpy as jnp
from jax import lax
from jax.experimental import pallas as pl
from jax.experimental.pallas import tpu as pltpu
from jax.experimental.pallas import tpu_sc as plsc

_NUM_ENTITIES = 1000000
_NUM_RELATIONS = 1000
_EMB_DIM = 64
_BATCH = 16384
_L = 16
_NC, _NS = 2, 16
_NW = _NC * _NS
_BPW = _BATCH // _NW
_CH = 16
_NCH = _BPW // _CH
_NP = _NCH // 2


def _rsqrt_nr(s):
    i = plsc.bitcast(s, jnp.int32)
    i = jnp.int32(0x5F3759DF) - lax.shift_right_logical(i, 1)
    y = plsc.bitcast(i, jnp.float32)
    for _ in range(2):
        y = y * (jnp.float32(1.5) - jnp.float32(0.5) * s * y * y)
    return y


def _make_sc_call():
    mesh = plsc.VectorSubcoreMesh(core_axis_name="c", subcore_axis_name="s")
    out = jax.ShapeDtypeStruct((_EMB_DIM, _BATCH), jnp.float32)

    @functools.partial(
        pl.kernel,
        mesh=mesh,
        out_type=[out, out, out],
        compiler_params=pltpu.CompilerParams(needs_layout_passes=False),
        scratch_types=[
            pltpu.VMEM((4, 128), jnp.int32),
            pltpu.VMEM((4, 128), jnp.int32),
            pltpu.VMEM((4, 128), jnp.int32),
            pltpu.VMEM((4, 128), jnp.int32),
            pltpu.VMEM((4, 128), jnp.int32),
            pltpu.VMEM((4, 128), jnp.int32),
            pltpu.VMEM((_CH, 8, _EMB_DIM), jnp.float32),
            pltpu.VMEM((_CH, 8, _EMB_DIM), jnp.float32),
            pltpu.VMEM((_EMB_DIM, _BPW), jnp.float32),
            pltpu.SemaphoreType.DMA,
            pltpu.SemaphoreType.DMA,
            pltpu.SemaphoreType.DMA,
        ],
    )
    def call(ent_hbm, rel_hbm, ph_hbm, pr_hbm, pt_hbm,
             h_out, r_out, t_out,
             raw_h, raw_r, raw_t, b8_h, b8_r, b8_t,
             blk0, blk1, outbuf, sem0, sem1, sem_out):
        wid = lax.axis_index("s") * _NC + lax.axis_index("c")
        base = wid * _BPW
        iota = jnp.arange(_L, dtype=jnp.int32)
        cols = [jnp.full((_L,), c, jnp.int32) for c in range(_EMB_DIM)]

        for pos_hbm, raw, b8 in ((ph_hbm, raw_h, b8_h),
                                 (pt_hbm, raw_t, b8_t),
                                 (pr_hbm, raw_r, b8_r)):
            for j in range(4):
                pltpu.sync_copy(pos_hbm.at[pl.ds(base + j * 128, 128)],
                                raw.at[j])
            for j in range(4):
                for o in range(0, 128, _L):
                    b8[j, pl.ds(o, _L)] = lax.shift_right_logical(
                        raw[j, pl.ds(o, _L)], 3)

        def fire(tbl, b8, c, blk, sem):
            j = lax.shift_right_logical(c, 3)
            o = (c & 7) * _L
            b8_vec = b8[j, pl.ds(o, _L)]
            for k in range(_CH):
                pltpu.async_copy(tbl.at[b8_vec[k]], blk.at[k], sem)

        def drain(tbl, blk, sem):
            pltpu.make_async_copy(tbl.at[pl.ds(0, _CH)], blk, sem).wait()

        def process(raw, c, blk, normalize):
            j = lax.shift_right_logical(c, 3)
            o = (c & 7) * _L
            idx_vec = raw[j, pl.ds(o, _L)]
            wr = idx_vec & 7
            dst = pl.ds(c * _L, _L)
            if normalize:
                acc0 = jnp.zeros((_L,), jnp.float32)
                acc1 = jnp.zeros((_L,), jnp.float32)
                acc2 = jnp.zeros((_L,), jnp.float32)
                acc3 = jnp.zeros((_L,), jnp.float32)
                for c4 in range(0, _EMB_DIM, 4):
                    v0 = plsc.load_gather(blk, [iota, wr, cols[c4]])
                    v1 = plsc.load_gather(blk, [iota, wr, cols[c4 + 1]])
                    v2 = plsc.load_gather(blk, [iota, wr, cols[c4 + 2]])
                    v3 = plsc.load_gather(blk, [iota, wr, cols[c4 + 3]])
                    acc0 = acc0 + v0 * v0
                    acc1 = acc1 + v1 * v1
                    acc2 = acc2 + v2 * v2
                    acc3 = acc3 + v3 * v3
                s = (acc0 + acc1) + (acc2 + acc3)
                y = _rsqrt_nr(s)
                y = jnp.where(idx_vec == _NUM_ENTITIES - 1,
                              jnp.float32(1.0), y)
                for cc in range(_EMB_DIM):
                    v = plsc.load_gather(blk, [iota, wr, cols[cc]])
                    outbuf[cc, dst] = v * y
            else:
                for cc in range(_EMB_DIM):
                    v = plsc.load_gather(blk, [iota, wr, cols[cc]])
                    outbuf[cc, dst] = v

        def run_tensor(tbl, raw, b8, normalize, pre=None):
            fire(tbl, b8, 0, blk0, sem0)
            fire(tbl, b8, 1, blk1, sem1)
            if pre is not None:
                pre.wait()

            def body(p, carry):
                c = p * 2
                drain(tbl, blk0, sem0)
                process(raw, c, blk0, normalize)

                @pl.when(p < _NP - 1)
                def _():
                    fire(tbl, b8, c + 2, blk0, sem0)

                drain(tbl, blk1, sem1)
                process(raw, c + 1, blk1, normalize)

                @pl.when(p < _NP - 1)
                def _():
                    fire(tbl, b8, c + 3, blk1, sem1)

                return carry

            lax.fori_loop(0, _NP, body, 0, unroll=False)

        out_slice = pl.ds(base, _BPW)
        run_tensor(ent_hbm, raw_h, b8_h, True)
        cp_h = pltpu.async_copy(outbuf, h_out.at[:, out_slice], sem_out)
        run_tensor(ent_hbm, raw_t, b8_t, True, pre=cp_h)
        cp_t = pltpu.async_copy(outbuf, t_out.at[:, out_slice], sem_out)
        run_tensor(rel_hbm, raw_r, b8_r, False, pre=cp_t)
        cp_r = pltpu.async_copy(outbuf, r_out.at[:, out_slice], sem_out)
        cp_r.wait()

    return call


_sc_call = _make_sc_call()


def kernel(entity_embds, rel_embds, pos_h, pos_r, pos_t):
    ent3 = jnp.reshape(entity_embds, (_NUM_ENTITIES // 8, 8, _EMB_DIM))
    rel3 = jnp.reshape(rel_embds, (_NUM_RELATIONS // 8, 8, _EMB_DIM))
    hT, rT, tT = _sc_call(ent3, rel3,
                          pos_h.astype(jnp.int32),
                          pos_r.astype(jnp.int32),
                          pos_t.astype(jnp.int32))
    return (hT.T, rT.T, tT.T)


# speedup vs baseline: 1.8333x; 1.0118x over previous
import functools

import jax
import jax.numpy as jnp
from jax import lax
from jax.experimental import pallas as pl
from jax.experimental.pallas import tpu as pltpu
from jax.experimental.pallas import tpu_sc as plsc

_NUM_ENTITIES = 1000000
_NUM_RELATIONS = 1000
_EMB_DIM = 64
_BATCH = 16384
_L = 16
_NC, _NS = 2, 16
_NW = _NC * _NS
_BPW = _BATCH // _NW
_CH = 16
_NCH = _BPW // _CH
_NP = _NCH // 2


def _rsqrt_nr(s):
    i = plsc.bitcast(s, jnp.int32)
    i = jnp.int32(0x5F3759DF) - lax.shift_right_logical(i, 1)
    y = plsc.bitcast(i, jnp.float32)
    for _ in range(2):
        y = y * (jnp.float32(1.5) - jnp.float32(0.5) * s * y * y)
    return y


def _make_sc_call():
    mesh = plsc.VectorSubcoreMesh(core_axis_name="c", subcore_axis_name="s")
    out = jax.ShapeDtypeStruct((_EMB_DIM, _BATCH), jnp.float32)

    @functools.partial(
        pl.kernel,
        mesh=mesh,
        out_type=[out, out, out],
        compiler_params=pltpu.CompilerParams(needs_layout_passes=False),
        scratch_types=[
            pltpu.VMEM((4, 128), jnp.int32),
            pltpu.VMEM((4, 128), jnp.int32),
            pltpu.VMEM((4, 128), jnp.int32),
            pltpu.VMEM((4, 128), jnp.int32),
            pltpu.VMEM((4, 128), jnp.int32),
            pltpu.VMEM((4, 128), jnp.int32),
            pltpu.VMEM((_CH, 8, _EMB_DIM), jnp.float32),
            pltpu.VMEM((_CH, 8, _EMB_DIM), jnp.float32),
            pltpu.VMEM((_EMB_DIM, _BPW), jnp.float32),
            pltpu.SemaphoreType.DMA,
            pltpu.SemaphoreType.DMA,
            pltpu.SemaphoreType.DMA,
        ],
    )
    def call(ent_hbm, rel_hbm, ph_hbm, pr_hbm, pt_hbm,
             h_out, r_out, t_out,
             raw_h, raw_r, raw_t, b8_h, b8_r, b8_t,
             blk0, blk1, outbuf, sem0, sem1, sem_out):
        wid = lax.axis_index("s") * _NC + lax.axis_index("c")
        base = wid * _BPW
        iota = jnp.arange(_L, dtype=jnp.int32)
        cols = [jnp.full((_L,), c, jnp.int32) for c in range(_EMB_DIM)]

        idx_cps = []
        for pos_hbm, raw in ((ph_hbm, raw_h), (pt_hbm, raw_t),
                             (pr_hbm, raw_r)):
            for j in range(4):
                idx_cps.append(pltpu.async_copy(
                    pos_hbm.at[pl.ds(base + j * 128, 128)], raw.at[j],
                    sem_out))
        for cp in idx_cps:
            cp.wait()
        for raw, b8 in ((raw_h, b8_h), (raw_t, b8_t), (raw_r, b8_r)):
            for j in range(4):
                for o in range(0, 128, _L):
                    b8[j, pl.ds(o, _L)] = lax.shift_right_logical(
                        raw[j, pl.ds(o, _L)], 3)

        def fire(tbl, b8, c, blk, sem):
            j = lax.shift_right_logical(c, 3)
            o = (c & 7) * _L
            b8_vec = b8[j, pl.ds(o, _L)]
            for k in range(_CH):
                pltpu.async_copy(tbl.at[b8_vec[k]], blk.at[k], sem)

        def drain(tbl, blk, sem):
            pltpu.make_async_copy(tbl.at[pl.ds(0, _CH)], blk, sem).wait()

        def process(raw, c, blk, normalize):
            j = lax.shift_right_logical(c, 3)
            o = (c & 7) * _L
            idx_vec = raw[j, pl.ds(o, _L)]
            wr = idx_vec & 7
            dst = pl.ds(c * _L, _L)
            if normalize:
                acc0 = jnp.zeros((_L,), jnp.float32)
                acc1 = jnp.zeros((_L,), jnp.float32)
                acc2 = jnp.zeros((_L,), jnp.float32)
                acc3 = jnp.zeros((_L,), jnp.float32)
                for c4 in range(0, _EMB_DIM, 4):
                    v0 = plsc.load_gather(blk, [iota, wr, cols[c4]])
                    v1 = plsc.load_gather(blk, [iota, wr, cols[c4 + 1]])
                    v2 = plsc.load_gather(blk, [iota, wr, cols[c4 + 2]])
                    v3 = plsc.load_gather(blk, [iota, wr, cols[c4 + 3]])
                    acc0 = acc0 + v0 * v0
                    acc1 = acc1 + v1 * v1
                    acc2 = acc2 + v2 * v2
                    acc3 = acc3 + v3 * v3
                s = (acc0 + acc1) + (acc2 + acc3)
                y = _rsqrt_nr(s)
                y = jnp.where(idx_vec == _NUM_ENTITIES - 1,
                              jnp.float32(1.0), y)
                for cc in range(_EMB_DIM):
                    v = plsc.load_gather(blk, [iota, wr, cols[cc]])
                    outbuf[cc, dst] = v * y
            else:
                for cc in range(_EMB_DIM):
                    v = plsc.load_gather(blk, [iota, wr, cols[cc]])
                    outbuf[cc, dst] = v

        def run_tensor(tbl, raw, b8, normalize, pre=None):
            fire(tbl, b8, 0, blk0, sem0)
            fire(tbl, b8, 1, blk1, sem1)
            if pre is not None:
                pre.wait()

            def body(p, carry):
                c = p * 2
                drain(tbl, blk0, sem0)
                process(raw, c, blk0, normalize)

                @pl.when(p < _NP - 1)
                def _():
                    fire(tbl, b8, c + 2, blk0, sem0)

                drain(tbl, blk1, sem1)
                process(raw, c + 1, blk1, normalize)

                @pl.when(p < _NP - 1)
                def _():
                    fire(tbl, b8, c + 3, blk1, sem1)

                return carry

            lax.fori_loop(0, _NP, body, 0, unroll=False)

        out_slice = pl.ds(base, _BPW)
        run_tensor(ent_hbm, raw_h, b8_h, True)
        cp_h = pltpu.async_copy(outbuf, h_out.at[:, out_slice], sem_out)
        run_tensor(ent_hbm, raw_t, b8_t, True, pre=cp_h)
        cp_t = pltpu.async_copy(outbuf, t_out.at[:, out_slice], sem_out)
        run_tensor(rel_hbm, raw_r, b8_r, False, pre=cp_t)
        cp_r = pltpu.async_copy(outbuf, r_out.at[:, out_slice], sem_out)
        cp_r.wait()

    return call


_sc_call = _make_sc_call()


def kernel(entity_embds, rel_embds, pos_h, pos_r, pos_t):
    ent3 = jnp.reshape(entity_embds, (_NUM_ENTITIES // 8, 8, _EMB_DIM))
    rel3 = jnp.reshape(rel_embds, (_NUM_RELATIONS // 8, 8, _EMB_DIM))
    hT, rT, tT = _sc_call(ent3, rel3,
                          pos_h.astype(jnp.int32),
                          pos_r.astype(jnp.int32),
                          pos_t.astype(jnp.int32))
    return (hT.T, rT.T, tT.T)
